# Initial kernel scaffold; baseline (speedup 1.0000x reference)
import jax
import jax.numpy as jnp
from jax.experimental import pallas as pl


def kernel(x, h, edge_index, W_enc, b_enc, W1, b1, W2, b2, W3, b3, Wt, bt, gat_W_0, gat_as_0, gat_ad_0, gat_b_0, gat_W_1, gat_as_1, gat_ad_1, gat_b_1, gat_W_2, gat_as_2, gat_ad_2, gat_b_2):
    raise NotImplementedError("write your pallas kernel here")


# speedup vs baseline: 19.4428x; 19.4428x over previous
import functools

import jax
import jax.numpy as jnp
from jax import lax
from jax.experimental import pallas as pl
from jax.experimental.pallas import tpu as pltpu
from jax.experimental.pallas import tpu_sc as plsc

N = 10000
NP = 10112
D = 128
E_TOT = 330000
NC, NS = 2, 16
NW = NC * NS
B = 128
NB = 81
PW = NB * B
E_PAD = NW * PW
RB = 128
GRID = NP // RB
RPT = NP // NS
DROW = NP // D
EPS = 1e-16


def _enc_body(xh, We, be, W0, as0, ad0, z_o, xl_o, als_o, ald_o):
    z = jnp.dot(xh[...], We[...], preferred_element_type=jnp.float32) + be[...]
    xl = jnp.dot(z, W0[...], preferred_element_type=jnp.float32)
    z_o[...] = z
    xl_o[...] = xl
    als_o[...] = jnp.sum(xl * as0[...], axis=1)[None, :]
    ald_o[...] = jnp.sum(xl * ad0[...], axis=1)[None, :]


def _enc(xh, We, be, W0, as0, ad0):
    return pl.pallas_call(
        _enc_body,
        grid=(GRID,),
        in_specs=[
            pl.BlockSpec((RB, 2 * D), lambda i: (i, 0)),
            pl.BlockSpec((2 * D, D), lambda i: (0, 0)),
            pl.BlockSpec((1, D), lambda i: (0, 0)),
            pl.BlockSpec((D, D), lambda i: (0, 0)),
            pl.BlockSpec((1, D), lambda i: (0, 0)),
            pl.BlockSpec((1, D), lambda i: (0, 0)),
        ],
        out_specs=[
            pl.BlockSpec((RB, D), lambda i: (i, 0)),
            pl.BlockSpec((RB, D), lambda i: (i, 0)),
            pl.BlockSpec((1, RB), lambda i: (0, i)),
            pl.BlockSpec((1, RB), lambda i: (0, i)),
        ],
        out_shape=[
            jax.ShapeDtypeStruct((NP, D), jnp.float32),
            jax.ShapeDtypeStruct((NP, D), jnp.float32),
            jax.ShapeDtypeStruct((1, NP), jnp.float32),
            jax.ShapeDtypeStruct((1, NP), jnp.float32),
        ],
    )(xh, We, be, W0, as0, ad0)


def _mid_body(num, den, bprev, W, asv, adv, xl_o, als_o, ald_o):
    den_sum = jnp.sum(den[...], axis=0)
    num_sum = num[0] + num[1]
    xx = num_sum / (den_sum + EPS)[:, None] + bprev[...]
    xx = jnp.maximum(xx, 0.0)
    xl = jnp.dot(xx, W[...], preferred_element_type=jnp.float32)
    xl_o[...] = xl
    als_o[...] = jnp.sum(xl * asv[...], axis=1)[None, :]
    ald_o[...] = jnp.sum(xl * adv[...], axis=1)[None, :]


def _mid(num, den, bprev, W, asv, adv):
    return pl.pallas_call(
        _mid_body,
        grid=(GRID,),
        in_specs=[
            pl.BlockSpec((NC, RB, D), lambda i: (0, i, 0)),
            pl.BlockSpec((NW, RB), lambda i: (0, i)),
            pl.BlockSpec((1, D), lambda i: (0, 0)),
            pl.BlockSpec((D, D), lambda i: (0, 0)),
            pl.BlockSpec((1, D), lambda i: (0, 0)),
            pl.BlockSpec((1, D), lambda i: (0, 0)),
        ],
        out_specs=[
            pl.BlockSpec((RB, D), lambda i: (i, 0)),
            pl.BlockSpec((1, RB), lambda i: (0, i)),
            pl.BlockSpec((1, RB), lambda i: (0, i)),
        ],
        out_shape=[
            jax.ShapeDtypeStruct((NP, D), jnp.float32),
            jax.ShapeDtypeStruct((1, NP), jnp.float32),
            jax.ShapeDtypeStruct((1, NP), jnp.float32),
        ],
    )(num, den, bprev, W, asv, adv)


def _fin_body(num, den, bg, z, W1a, W1b, b1, W2, b2, W3p, b3p, Wtp, btp,
              hh_o, y_o, t_o, acc):
    i = pl.program_id(0)
    den_sum = jnp.sum(den[...], axis=0)
    num_sum = num[0] + num[1]
    hh = num_sum / (den_sum + EPS)[:, None] + bg[...]
    hh_o[...] = hh
    o = jnp.dot(hh, W1a[...], preferred_element_type=jnp.float32)
    o = o + jnp.dot(z[...], W1b[...], preferred_element_type=jnp.float32)
    o = jnp.maximum(o + b1[...], 0.0)
    o = jnp.maximum(jnp.dot(o, W2[...], preferred_element_type=jnp.float32) + b2[...], 0.0)
    y_o[...] = jax.nn.sigmoid(
        jnp.dot(o, W3p[...], preferred_element_type=jnp.float32) + b3p[...])

    @pl.when(i == 0)
    def _():
        acc[...] = jnp.zeros_like(acc)

    rows = i * RB + lax.broadcasted_iota(jnp.int32, (RB, 1), 0)
    hm = jnp.where(rows < N, hh, 0.0)
    acc[...] = acc[...] + jnp.sum(hm, axis=0)[None, :]

    @pl.when(i == GRID - 1)
    def _():
        mean = acc[...] * (1.0 / N)
        t_o[...] = jax.nn.sigmoid(
            jnp.dot(mean, Wtp[...], preferred_element_type=jnp.float32) + btp[...])


def _fin(num, den, bg, z, W1a, W1b, b1, W2, b2, W3p, b3p, Wtp, btp):
    return pl.pallas_call(
        _fin_body,
        grid=(GRID,),
        in_specs=[
            pl.BlockSpec((NC, RB, D), lambda i: (0, i, 0)),
            pl.BlockSpec((NW, RB), lambda i: (0, i)),
            pl.BlockSpec((1, D), lambda i: (0, 0)),
            pl.BlockSpec((RB, D), lambda i: (i, 0)),
            pl.BlockSpec((D, D), lambda i: (0, 0)),
            pl.BlockSpec((D, D), lambda i: (0, 0)),
            pl.BlockSpec((1, D), lambda i: (0, 0)),
            pl.BlockSpec((D, D), lambda i: (0, 0)),
            pl.BlockSpec((1, D), lambda i: (0, 0)),
            pl.BlockSpec((D, D), lambda i: (0, 0)),
            pl.BlockSpec((1, D), lambda i: (0, 0)),
            pl.BlockSpec((D, D), lambda i: (0, 0)),
            pl.BlockSpec((1, D), lambda i: (0, 0)),
        ],
        out_specs=[
            pl.BlockSpec((RB, D), lambda i: (i, 0)),
            pl.BlockSpec((RB, D), lambda i: (i, 0)),
            pl.BlockSpec((1, D), lambda i: (0, 0)),
        ],
        out_shape=[
            jax.ShapeDtypeStruct((NP, D), jnp.float32),
            jax.ShapeDtypeStruct((NP, D), jnp.float32),
            jax.ShapeDtypeStruct((1, D), jnp.float32),
        ],
        scratch_shapes=[pltpu.VMEM((1, D), jnp.float32)],
    )(num, den, bg, z, W1a, W1b, b1, W2, b2, W3p, b3p, Wtp, btp)


def _edge_body(xl_hbm, als_hbm, ald_hbm, sd_hbm,
               num_hbm, den_hbm,
               als_v, ald_v, ring, rows_v, ex_v, den_v, num_sh, sem):
    c = lax.axis_index("c")
    s = lax.axis_index("s")
    wid = c * NS + s

    pltpu.sync_copy(als_hbm, als_v)
    pltpu.sync_copy(ald_hbm, ald_v)

    zero16 = jnp.zeros((16,), jnp.float32)

    def _zden(i, carry):
        r = i // 8
        k = i % 8
        den_v[r, pl.ds(pl.multiple_of(k * 16, 16), 16)] = zero16
        return carry
    lax.fori_loop(0, DROW * 8, _zden, 0)

    def _zrows(i, carry):
        r = i // 8
        k = i % 8
        rows_v[r, pl.ds(pl.multiple_of(k * 16, 16), 16)] = zero16
        return carry
    lax.fori_loop(0, B * 8, _zrows, 0)

    row0 = s * RPT
    for k in range(4):
        pltpu.sync_copy(rows_v, num_sh.at[pl.ds(row0 + k * B, B)])
    pltpu.sync_copy(rows_v.at[pl.ds(0, RPT - 4 * B)],
                    num_sh.at[pl.ds(row0 + 4 * B, RPT - 4 * B)])

    plsc.subcore_barrier()

    def _batch(j, carry):
        pltpu.sync_copy(sd_hbm.at[wid, j], ring)
        pltpu.async_copy(xl_hbm.at[ring.at[0]], rows_v, sem).wait()
        for r16 in range(B // 16):
            sl = pl.ds(r16 * 16, 16)
            si = ring[0, sl]
            di = ring[1, sl]
            a = plsc.load_gather(als_v, [si]) + plsc.load_gather(ald_v, [di])
            a = jnp.where(a >= 0.0, a, a * 0.2)
            ex = jnp.exp(a)
            plsc.addupdate_scatter(den_v, [di >> 7, di & 127], ex)
            ex_v[sl] = ex

        def _scale(r, inner):
            e = ex_v[pl.ds(r, 16)][0]
            for k in range(D // 16):
                cs = pl.ds(k * 16, 16)
                rows_v[r, cs] = rows_v[r, cs] * e
            return inner
        lax.fori_loop(0, B, _scale, 0)

        pltpu.sync_copy(rows_v, num_sh.at[ring.at[1]], add=True)
        return carry
    lax.fori_loop(0, NB, _batch, 0)

    plsc.subcore_barrier()
    pltpu.sync_copy(num_sh.at[pl.ds(row0, RPT)], num_hbm.at[c, pl.ds(row0, RPT)])
    pltpu.sync_copy(den_v, den_hbm.at[wid])


_edge = pl.kernel(
    _edge_body,
    out_type=[
        jax.ShapeDtypeStruct((NC, NP, D), jnp.float32),
        jax.ShapeDtypeStruct((NW, DROW, D), jnp.float32),
    ],
    mesh=plsc.VectorSubcoreMesh(core_axis_name="c", subcore_axis_name="s",
                                num_cores=NC, num_subcores=NS),
    compiler_params=pltpu.CompilerParams(needs_layout_passes=False),
    scratch_types=[
        pltpu.VMEM((NP,), jnp.float32),
        pltpu.VMEM((NP,), jnp.float32),
        pltpu.VMEM((2, B), jnp.int32),
        pltpu.VMEM((B, D), jnp.float32),
        pltpu.VMEM((B + 16,), jnp.float32),
        pltpu.VMEM((DROW, D), jnp.float32),
        pltpu.VMEM_SHARED((NP, D), jnp.float32),
        pltpu.SemaphoreType.DMA,
    ],
)


def kernel(x, h, edge_index, W_enc, b_enc, W1, b1, W2, b2, W3, b3, Wt, bt,
           gat_W_0, gat_as_0, gat_ad_0, gat_b_0,
           gat_W_1, gat_as_1, gat_ad_1, gat_b_1,
           gat_W_2, gat_as_2, gat_ad_2, gat_b_2):
    f32 = jnp.float32
    r1 = lambda v: v.reshape(1, D)
    xh = jnp.zeros((NP, 2 * D), f32).at[:N, :D].set(x).at[:N, D:].set(h)
    idx = jnp.arange(N, dtype=jnp.int32)
    pad = jnp.full((E_PAD - E_TOT,), N, jnp.int32)
    src = jnp.concatenate([edge_index[0], idx, pad]).reshape(NW, NB, 1, B)
    dst = jnp.concatenate([edge_index[1], idx, pad]).reshape(NW, NB, 1, B)
    sd = jnp.concatenate([src, dst], axis=2)

    z, xl, als, ald = _enc(xh, W_enc, r1(b_enc), gat_W_0, r1(gat_as_0), r1(gat_ad_0))
    num, den = _edge(xl, als.reshape(NP), ald.reshape(NP), sd)
    xl, als, ald = _mid(num, den.reshape(NW, NP), r1(gat_b_0), gat_W_1, r1(gat_as_1), r1(gat_ad_1))
    num, den = _edge(xl, als.reshape(NP), ald.reshape(NP), sd)
    xl, als, ald = _mid(num, den.reshape(NW, NP), r1(gat_b_1), gat_W_2, r1(gat_as_2), r1(gat_ad_2))
    num, den = _edge(xl, als.reshape(NP), ald.reshape(NP), sd)
    xl, als, ald = _mid(num, den.reshape(NW, NP), r1(gat_b_2), gat_W_2, r1(gat_as_2), r1(gat_ad_2))
    num, den = _edge(xl, als.reshape(NP), ald.reshape(NP), sd)

    W3p = jnp.zeros((D, D), f32).at[:, :1].set(W3)
    b3p = jnp.zeros((1, D), f32).at[0, 0].set(b3[0])
    Wtp = jnp.zeros((D, D), f32).at[:, :1].set(Wt)
    btp = jnp.zeros((1, D), f32).at[0, 0].set(bt[0])
    hh_f, y_f, t_f = _fin(num, den.reshape(NW, NP), r1(gat_b_2), z, W1[:D], W1[D:],
                          r1(b1), W2, r1(b2), W3p, b3p, Wtp, btp)
    return (y_f[:N, :1], t_f[0, :1], hh_f[:N])


# speedup vs baseline: 26.3495x; 1.3552x over previous
import functools

import jax
import jax.numpy as jnp
from jax import lax
from jax.experimental import pallas as pl
from jax.experimental.pallas import tpu as pltpu
from jax.experimental.pallas import tpu_sc as plsc

N = 10000
NP = 10112
D = 128
E_TOT = 330000
NC, NS = 2, 16
NW = NC * NS
B = 64
NB = 162
PW = NB * B
E_PAD = NW * PW
RB = 128
GRID = NP // RB
RPT = NP // NS
DROW = NP // D
EPS = 1e-16


def _enc_body(xh, We, be, W0, as0, ad0, z_o, xl_o, als_o, ald_o):
    z = jnp.dot(xh[...], We[...], preferred_element_type=jnp.float32) + be[...]
    xl = jnp.dot(z, W0[...], preferred_element_type=jnp.float32)
    z_o[...] = z
    xl_o[...] = xl
    als_o[...] = jnp.sum(xl * as0[...], axis=1)[None, :]
    ald_o[...] = jnp.sum(xl * ad0[...], axis=1)[None, :]


def _enc(xh, We, be, W0, as0, ad0):
    return pl.pallas_call(
        _enc_body,
        grid=(GRID,),
        in_specs=[
            pl.BlockSpec((RB, 2 * D), lambda i: (i, 0)),
            pl.BlockSpec((2 * D, D), lambda i: (0, 0)),
            pl.BlockSpec((1, D), lambda i: (0, 0)),
            pl.BlockSpec((D, D), lambda i: (0, 0)),
            pl.BlockSpec((1, D), lambda i: (0, 0)),
            pl.BlockSpec((1, D), lambda i: (0, 0)),
        ],
        out_specs=[
            pl.BlockSpec((RB, D), lambda i: (i, 0)),
            pl.BlockSpec((RB, D), lambda i: (i, 0)),
            pl.BlockSpec((1, RB), lambda i: (0, i)),
            pl.BlockSpec((1, RB), lambda i: (0, i)),
        ],
        out_shape=[
            jax.ShapeDtypeStruct((NP, D), jnp.float32),
            jax.ShapeDtypeStruct((NP, D), jnp.float32),
            jax.ShapeDtypeStruct((1, NP), jnp.float32),
            jax.ShapeDtypeStruct((1, NP), jnp.float32),
        ],
    )(xh, We, be, W0, as0, ad0)


def _mid_body(num, den, bprev, W, asv, adv, xl_o, als_o, ald_o):
    den_sum = jnp.sum(den[...], axis=0)
    num_sum = num[0] + num[1]
    xx = num_sum / (den_sum + EPS)[:, None] + bprev[...]
    xx = jnp.maximum(xx, 0.0)
    xl = jnp.dot(xx, W[...], preferred_element_type=jnp.float32)
    xl_o[...] = xl
    als_o[...] = jnp.sum(xl * asv[...], axis=1)[None, :]
    ald_o[...] = jnp.sum(xl * adv[...], axis=1)[None, :]


def _mid(num, den, bprev, W, asv, adv):
    return pl.pallas_call(
        _mid_body,
        grid=(GRID,),
        in_specs=[
            pl.BlockSpec((NC, RB, D), lambda i: (0, i, 0)),
            pl.BlockSpec((NW, RB), lambda i: (0, i)),
            pl.BlockSpec((1, D), lambda i: (0, 0)),
            pl.BlockSpec((D, D), lambda i: (0, 0)),
            pl.BlockSpec((1, D), lambda i: (0, 0)),
            pl.BlockSpec((1, D), lambda i: (0, 0)),
        ],
        out_specs=[
            pl.BlockSpec((RB, D), lambda i: (i, 0)),
            pl.BlockSpec((1, RB), lambda i: (0, i)),
            pl.BlockSpec((1, RB), lambda i: (0, i)),
        ],
        out_shape=[
            jax.ShapeDtypeStruct((NP, D), jnp.float32),
            jax.ShapeDtypeStruct((1, NP), jnp.float32),
            jax.ShapeDtypeStruct((1, NP), jnp.float32),
        ],
    )(num, den, bprev, W, asv, adv)


def _fin_body(num, den, bg, z, W1a, W1b, b1, W2, b2, W3p, b3p, Wtp, btp,
              hh_o, y_o, t_o, acc):
    i = pl.program_id(0)
    den_sum = jnp.sum(den[...], axis=0)
    num_sum = num[0] + num[1]
    hh = num_sum / (den_sum + EPS)[:, None] + bg[...]
    hh_o[...] = hh
    o = jnp.dot(hh, W1a[...], preferred_element_type=jnp.float32)
    o = o + jnp.dot(z[...], W1b[...], preferred_element_type=jnp.float32)
    o = jnp.maximum(o + b1[...], 0.0)
    o = jnp.maximum(jnp.dot(o, W2[...], preferred_element_type=jnp.float32) + b2[...], 0.0)
    y_o[...] = jax.nn.sigmoid(
        jnp.dot(o, W3p[...], preferred_element_type=jnp.float32) + b3p[...])

    @pl.when(i == 0)
    def _():
        acc[...] = jnp.zeros_like(acc)

    rows = i * RB + lax.broadcasted_iota(jnp.int32, (RB, 1), 0)
    hm = jnp.where(rows < N, hh, 0.0)
    acc[...] = acc[...] + jnp.sum(hm, axis=0)[None, :]

    @pl.when(i == GRID - 1)
    def _():
        mean = acc[...] * (1.0 / N)
        t_o[...] = jax.nn.sigmoid(
            jnp.dot(mean, Wtp[...], preferred_element_type=jnp.float32) + btp[...])


def _fin(num, den, bg, z, W1a, W1b, b1, W2, b2, W3p, b3p, Wtp, btp):
    return pl.pallas_call(
        _fin_body,
        grid=(GRID,),
        in_specs=[
            pl.BlockSpec((NC, RB, D), lambda i: (0, i, 0)),
            pl.BlockSpec((NW, RB), lambda i: (0, i)),
            pl.BlockSpec((1, D), lambda i: (0, 0)),
            pl.BlockSpec((RB, D), lambda i: (i, 0)),
            pl.BlockSpec((D, D), lambda i: (0, 0)),
            pl.BlockSpec((D, D), lambda i: (0, 0)),
            pl.BlockSpec((1, D), lambda i: (0, 0)),
            pl.BlockSpec((D, D), lambda i: (0, 0)),
            pl.BlockSpec((1, D), lambda i: (0, 0)),
            pl.BlockSpec((D, D), lambda i: (0, 0)),
            pl.BlockSpec((1, D), lambda i: (0, 0)),
            pl.BlockSpec((D, D), lambda i: (0, 0)),
            pl.BlockSpec((1, D), lambda i: (0, 0)),
        ],
        out_specs=[
            pl.BlockSpec((RB, D), lambda i: (i, 0)),
            pl.BlockSpec((RB, D), lambda i: (i, 0)),
            pl.BlockSpec((1, D), lambda i: (0, 0)),
        ],
        out_shape=[
            jax.ShapeDtypeStruct((NP, D), jnp.float32),
            jax.ShapeDtypeStruct((NP, D), jnp.float32),
            jax.ShapeDtypeStruct((1, D), jnp.float32),
        ],
        scratch_shapes=[pltpu.VMEM((1, D), jnp.float32)],
    )(num, den, bg, z, W1a, W1b, b1, W2, b2, W3p, b3p, Wtp, btp)


def _edge_body(xl_hbm, als_hbm, ald_hbm, sd_hbm,
               num_hbm, den_hbm,
               als_v, ald_v, ring, rows2, ex_v, den_v, num_sh,
               sem_r, sem_g, sem_sc):
    c = lax.axis_index("c")
    s = lax.axis_index("s")
    wid = c * NS + s

    pltpu.sync_copy(als_hbm, als_v)
    pltpu.sync_copy(ald_hbm, ald_v)

    zero16 = jnp.zeros((16,), jnp.float32)

    def _zden(i, carry):
        r = i // 8
        k = i % 8
        den_v[r, pl.ds(pl.multiple_of(k * 16, 16), 16)] = zero16
        return carry
    lax.fori_loop(0, DROW * 8, _zden, 0)

    def _zrows(i, carry):
        r = i // 8
        k = i % 8
        rows2[0, r, pl.ds(pl.multiple_of(k * 16, 16), 16)] = zero16
        return carry
    lax.fori_loop(0, B * 8, _zrows, 0)

    row0 = s * RPT
    for k in range(9):
        pltpu.sync_copy(rows2.at[0], num_sh.at[pl.ds(row0 + k * B, B)])
    pltpu.sync_copy(rows2.at[0, pl.ds(0, RPT - 9 * B)],
                    num_sh.at[pl.ds(row0 + 9 * B, RPT - 9 * B)])

    plsc.subcore_barrier()

    pltpu.sync_copy(sd_hbm.at[wid, 0], ring.at[0])
    pltpu.sync_copy(sd_hbm.at[wid, 1], ring.at[1])
    pltpu.async_copy(xl_hbm.at[ring.at[0, 0]], rows2.at[0], sem_g.at[0])

    def _batch(j, carry):
        s2 = j & 1
        o2 = 1 - s2
        s4 = j & 3

        @pl.when(j + 2 < NB)
        def _():
            pltpu.async_copy(sd_hbm.at[wid, j + 2], ring.at[(j + 2) & 3],
                             sem_r.at[(j + 2) & 3])

        @pl.when(j >= 1)
        def _():
            pltpu.make_async_copy(rows2.at[o2],
                                  num_sh.at[ring.at[(j - 1) & 3, 1]],
                                  sem_sc.at[o2]).wait()

        @pl.when(j + 1 < NB)
        def _():
            @pl.when(j + 1 >= 2)
            def _():
                pltpu.make_async_copy(sd_hbm.at[wid, j + 1],
                                      ring.at[(j + 1) & 3],
                                      sem_r.at[(j + 1) & 3]).wait()
            pltpu.async_copy(xl_hbm.at[ring.at[(j + 1) & 3, 0]],
                             rows2.at[o2], sem_g.at[o2])

        pltpu.make_async_copy(xl_hbm.at[ring.at[s4, 0]], rows2.at[s2],
                              sem_g.at[s2]).wait()
        for r16 in range(B // 16):
            sl = pl.ds(r16 * 16, 16)
            si = ring[s4, 0, sl]
            di = ring[s4, 1, sl]
            a = plsc.load_gather(als_v, [si]) + plsc.load_gather(ald_v, [di])
            a = jnp.where(a >= 0.0, a, a * 0.2)
            ex = jnp.exp(a)
            plsc.addupdate_scatter(den_v, [di >> 7, di & 127], ex)
            ex_v[sl] = ex

        def _scale(r, inner):
            e = ex_v[pl.ds(r, 16)][0]
            for k in range(D // 16):
                cs = pl.ds(k * 16, 16)
                rows2[s2, r, cs] = rows2[s2, r, cs] * e
            return inner
        lax.fori_loop(0, B, _scale, 0)

        pltpu.async_copy(rows2.at[s2], num_sh.at[ring.at[s4, 1]],
                         sem_sc.at[s2], add=True)
        return carry
    lax.fori_loop(0, NB, _batch, 0)

    pltpu.make_async_copy(rows2.at[(NB - 1) & 1],
                          num_sh.at[ring.at[(NB - 1) & 3, 1]],
                          sem_sc.at[(NB - 1) & 1]).wait()

    plsc.subcore_barrier()
    pltpu.sync_copy(num_sh.at[pl.ds(row0, RPT)], num_hbm.at[c, pl.ds(row0, RPT)])
    pltpu.sync_copy(den_v, den_hbm.at[wid])


_edge = pl.kernel(
    _edge_body,
    out_type=[
        jax.ShapeDtypeStruct((NC, NP, D), jnp.float32),
        jax.ShapeDtypeStruct((NW, DROW, D), jnp.float32),
    ],
    mesh=plsc.VectorSubcoreMesh(core_axis_name="c", subcore_axis_name="s",
                                num_cores=NC, num_subcores=NS),
    compiler_params=pltpu.CompilerParams(needs_layout_passes=False),
    scratch_types=[
        pltpu.VMEM((NP,), jnp.float32),
        pltpu.VMEM((NP,), jnp.float32),
        pltpu.VMEM((4, 2, B), jnp.int32),
        pltpu.VMEM((2, B, D), jnp.float32),
        pltpu.VMEM((B + 16,), jnp.float32),
        pltpu.VMEM((DROW, D), jnp.float32),
        pltpu.VMEM_SHARED((NP, D), jnp.float32),
        pltpu.SemaphoreType.DMA((4,)),
        pltpu.SemaphoreType.DMA((2,)),
        pltpu.SemaphoreType.DMA((2,)),
    ],
)

def kernel(x, h, edge_index, W_enc, b_enc, W1, b1, W2, b2, W3, b3, Wt, bt,
           gat_W_0, gat_as_0, gat_ad_0, gat_b_0,
           gat_W_1, gat_as_1, gat_ad_1, gat_b_1,
           gat_W_2, gat_as_2, gat_ad_2, gat_b_2):
    f32 = jnp.float32
    r1 = lambda v: v.reshape(1, D)
    xh = jnp.zeros((NP, 2 * D), f32).at[:N, :D].set(x).at[:N, D:].set(h)
    idx = jnp.arange(N, dtype=jnp.int32)
    pad = jnp.full((E_PAD - E_TOT,), N, jnp.int32)
    src = jnp.concatenate([edge_index[0], idx, pad]).reshape(NW, NB, 1, B)
    dst = jnp.concatenate([edge_index[1], idx, pad]).reshape(NW, NB, 1, B)
    sd = jnp.concatenate([src, dst], axis=2)

    z, xl, als, ald = _enc(xh, W_enc, r1(b_enc), gat_W_0, r1(gat_as_0), r1(gat_ad_0))
    num, den = _edge(xl, als.reshape(NP), ald.reshape(NP), sd)
    xl, als, ald = _mid(num, den.reshape(NW, NP), r1(gat_b_0), gat_W_1, r1(gat_as_1), r1(gat_ad_1))
    num, den = _edge(xl, als.reshape(NP), ald.reshape(NP), sd)
    xl, als, ald = _mid(num, den.reshape(NW, NP), r1(gat_b_1), gat_W_2, r1(gat_as_2), r1(gat_ad_2))
    num, den = _edge(xl, als.reshape(NP), ald.reshape(NP), sd)
    xl, als, ald = _mid(num, den.reshape(NW, NP), r1(gat_b_2), gat_W_2, r1(gat_as_2), r1(gat_ad_2))
    num, den = _edge(xl, als.reshape(NP), ald.reshape(NP), sd)

    W3p = jnp.zeros((D, D), f32).at[:, :1].set(W3)
    b3p = jnp.zeros((1, D), f32).at[0, 0].set(b3[0])
    Wtp = jnp.zeros((D, D), f32).at[:, :1].set(Wt)
    btp = jnp.zeros((1, D), f32).at[0, 0].set(bt[0])
    hh_f, y_f, t_f = _fin(num, den.reshape(NW, NP), r1(gat_b_2), z, W1[:D], W1[D:],
                          r1(b1), W2, r1(b2), W3p, b3p, Wtp, btp)
    return (y_f[:N, :1], t_f[0, :1], hh_f[:N])
